# baseline (speedup 1.0000x reference)
import functools

import jax
import jax.numpy as jnp
from jax import lax
from jax.experimental import pallas as pl
from jax.experimental.pallas import tpu as pltpu
from jax.experimental.pallas import tpu_sc as plsc

N_NODES = 10000
N_EDGES = 320000
D_IN = 128
W = 16

NC = 2
NS = 16
NW = NC * NS

NP = 10240
ROWS_PER_TILE = NP // NS

CHUNK = 64
C0 = 128
C1 = 186
CMAX = max(C0, C1)
DUMMY = N_NODES

NBUF = 8
KAHEAD = 6

TCB = 2048
NTCB = NP // TCB


def _tc1_body(x_ref, w_ref, b_ref, o_ref):
    o_ref[...] = jnp.maximum(
        jnp.dot(x_ref[...], w_ref[...], preferred_element_type=jnp.float32)
        + b_ref[...], 0.0)


def _tc1(x_pad, w1e, b1e):
    return pl.pallas_call(
        _tc1_body,
        grid=(NTCB,),
        in_specs=[
            pl.BlockSpec((TCB, D_IN), lambda i: (i, 0)),
            pl.BlockSpec((D_IN, W), lambda i: (0, 0)),
            pl.BlockSpec((1, W), lambda i: (0, 0)),
        ],
        out_specs=pl.BlockSpec((TCB, W), lambda i: (i, 0)),
        out_shape=jax.ShapeDtypeStruct((NP, W), jnp.float32),
    )(x_pad, w1e, b1e)


def _tc2_body(acc_ref, deg_ref, hext_ref, wl_ref, bl_ref, wr_ref,
              h1_ref, dgc_ref):
    a = acc_ref[0] + acc_ref[1]
    dgc = jnp.maximum(deg_ref[0] + deg_ref[1], 1.0).reshape(TCB, 1)
    mean = a / dgc
    h1_ref[...] = (
        jnp.dot(mean, wl_ref[...], preferred_element_type=jnp.float32)
        + bl_ref[...]
        + jnp.dot(hext_ref[...], wr_ref[...],
                  preferred_element_type=jnp.float32))
    dgc_ref[...] = jnp.broadcast_to(dgc, (TCB, W))


def _tc2(acc1, deg, hext, wl1, bl1, wr1):
    return pl.pallas_call(
        _tc2_body,
        grid=(NTCB,),
        in_specs=[
            pl.BlockSpec((NC, TCB, W), lambda i: (0, i, 0)),
            pl.BlockSpec((NC, TCB), lambda i: (0, i)),
            pl.BlockSpec((TCB, W), lambda i: (i, 0)),
            pl.BlockSpec((W, W), lambda i: (0, 0)),
            pl.BlockSpec((1, W), lambda i: (0, 0)),
            pl.BlockSpec((W, W), lambda i: (0, 0)),
        ],
        out_specs=[
            pl.BlockSpec((TCB, W), lambda i: (i, 0)),
            pl.BlockSpec((TCB, W), lambda i: (i, 0)),
        ],
        out_shape=[
            jax.ShapeDtypeStruct((NP, W), jnp.float32),
            jax.ShapeDtypeStruct((NP, W), jnp.float32),
        ],
    )(acc1, deg, hext, wl1, bl1, wr1)


def _tc3_body(acc_ref, dgc_ref, h1_ref, wl_ref, bl_ref, wr_ref, w2_ref,
              b2_ref, o_ref):
    a = acc_ref[0] + acc_ref[1]
    mean = a / dgc_ref[...]
    h2 = (jnp.dot(mean, wl_ref[...], preferred_element_type=jnp.float32)
          + bl_ref[...]
          + jnp.dot(h1_ref[...], wr_ref[...],
                    preferred_element_type=jnp.float32))
    o_ref[...] = (jnp.dot(h2, w2_ref[...], preferred_element_type=jnp.float32)
                  + b2_ref[...])


def _tc3(acc2, dgc, h1, wl2, bl2, wr2, w2e, b2e):
    return pl.pallas_call(
        _tc3_body,
        grid=(NTCB,),
        in_specs=[
            pl.BlockSpec((NC, TCB, W), lambda i: (0, i, 0)),
            pl.BlockSpec((TCB, W), lambda i: (i, 0)),
            pl.BlockSpec((TCB, W), lambda i: (i, 0)),
            pl.BlockSpec((W, W), lambda i: (0, 0)),
            pl.BlockSpec((1, W), lambda i: (0, 0)),
            pl.BlockSpec((W, W), lambda i: (0, 0)),
            pl.BlockSpec((W, 8), lambda i: (0, 0)),
            pl.BlockSpec((1, 8), lambda i: (0, 0)),
        ],
        out_specs=pl.BlockSpec((TCB, 8), lambda i: (i, 0)),
        out_shape=jax.ShapeDtypeStruct((NP, 8), jnp.float32),
    )(acc2, dgc, h1, wl2, bl2, wr2, w2e, b2e)


def _sc_agg_body(with_deg, tab_hbm, src_hbm, dst_hbm, *rest):
    if with_deg:
        (out_hbm, dout_hbm, src_v, dst_v, rows_v, ones_v, zdeg_v, zrow_v,
         acc_sh, deg_sh, gsem, ssem, dsem, isem) = rest
    else:
        (out_hbm, src_v, dst_v, rows_v, zrow_v,
         acc_sh, gsem, ssem, isem) = rest
    cid = lax.axis_index("c")
    sid = lax.axis_index("s")
    wid = cid * NS + sid

    pltpu.async_copy(src_hbm.at[wid], src_v, isem).wait()
    pltpu.async_copy(dst_hbm.at[wid], dst_v, isem).wait()

    def _z(i, c):
        zrow_v[i] = jnp.zeros((W,), jnp.float32)
        return c
    lax.fori_loop(0, ROWS_PER_TILE, _z, 0)
    if with_deg:
        def _z1(i, c):
            zdeg_v[pl.ds(i * W, W)] = jnp.zeros((W,), jnp.float32)
            return c
        lax.fori_loop(0, ROWS_PER_TILE // W, _z1, 0)
        for i in range(CHUNK // W):
            ones_v[pl.ds(i * W, W)] = jnp.ones((W,), jnp.float32)
    row0 = sid * ROWS_PER_TILE
    pltpu.sync_copy(zrow_v, acc_sh.at[pl.ds(row0, ROWS_PER_TILE)])
    if with_deg:
        pltpu.sync_copy(zdeg_v, deg_sh.at[pl.ds(row0, ROWS_PER_TILE)])
    plsc.subcore_barrier()

    def _gather(j, b):
        pltpu.async_copy(tab_hbm.at[src_v.at[j]], rows_v.at[b], gsem.at[b])

    def _gather_wait(b):
        pltpu.make_async_copy(tab_hbm.at[src_v.at[0]], rows_v.at[b],
                              gsem.at[b]).wait()

    def _scatter(j, b):
        pltpu.async_copy(rows_v.at[b], acc_sh.at[dst_v.at[j]],
                         ssem.at[b], add=True)
        if with_deg:
            pltpu.async_copy(ones_v, deg_sh.at[dst_v.at[j]],
                             dsem.at[b], add=True)

    def _scatter_wait(b):
        pltpu.make_async_copy(rows_v.at[b], acc_sh.at[dst_v.at[0]],
                              ssem.at[b]).wait()
        if with_deg:
            pltpu.make_async_copy(ones_v, deg_sh.at[dst_v.at[0]],
                                  dsem.at[b]).wait()

    nche = lax.select(cid == 0, C0, C1)
    for j in range(KAHEAD):
        _gather(j, j)

    def _step(j, c):
        b = lax.rem(j, NBUF)
        jf = j + KAHEAD
        bf = lax.rem(jf, NBUF)

        @pl.when(jf < nche)
        def _prefetch():
            @pl.when(jf >= NBUF)
            def _drain():
                _scatter_wait(bf)
            _gather(jf, bf)

        _gather_wait(b)
        _scatter(j, b)
        return c

    lax.fori_loop(0, nche, _step, 0)

    for t in range(NBUF):
        _scatter_wait(t)
    plsc.subcore_barrier()
    pltpu.sync_copy(acc_sh.at[pl.ds(row0, ROWS_PER_TILE)],
                    out_hbm.at[cid, pl.ds(row0, ROWS_PER_TILE)])
    if with_deg:
        pltpu.sync_copy(deg_sh.at[pl.ds(row0, ROWS_PER_TILE)],
                        dout_hbm.at[cid, pl.ds(row0, ROWS_PER_TILE)])


def _sc_agg(table, srcp, dstp, with_deg):
    mesh = plsc.VectorSubcoreMesh(core_axis_name="c", subcore_axis_name="s")
    acc_t = jax.ShapeDtypeStruct((NC, NP, W), jnp.float32)
    if with_deg:
        out_type = (acc_t, jax.ShapeDtypeStruct((NC, NP), jnp.float32))
    else:
        out_type = acc_t
    scratch = [
        pltpu.VMEM((CMAX, CHUNK), jnp.int32),
        pltpu.VMEM((CMAX, CHUNK), jnp.int32),
        pltpu.VMEM((NBUF, CHUNK, W), jnp.float32),
    ]
    if with_deg:
        scratch.append(pltpu.VMEM((CHUNK,), jnp.float32))
        scratch.append(pltpu.VMEM((ROWS_PER_TILE,), jnp.float32))
    scratch.append(pltpu.VMEM((ROWS_PER_TILE, W), jnp.float32))
    scratch.append(pltpu.VMEM_SHARED((NP, W), jnp.float32))
    if with_deg:
        scratch.append(pltpu.VMEM_SHARED((NP,), jnp.float32))
    scratch.append(pltpu.SemaphoreType.DMA((NBUF,)))
    scratch.append(pltpu.SemaphoreType.DMA((NBUF,)))
    if with_deg:
        scratch.append(pltpu.SemaphoreType.DMA((NBUF,)))
    scratch.append(pltpu.SemaphoreType.DMA)
    kfn = pl.kernel(
        functools.partial(_sc_agg_body, with_deg),
        out_type=out_type,
        mesh=mesh,
        scratch_types=scratch,
        compiler_params=pltpu.CompilerParams(use_tc_tiling_on_sc=False),
    )
    return kfn(table, srcp, dstp)


def kernel(x, edge_index, W1, b1, Wl1, bl1, Wr1, Wl2, bl2, Wr2, W2, b2):
    f32 = jnp.float32
    x_pad = jnp.zeros((NP, D_IN), f32).at[:N_NODES].set(x)

    w1e = jnp.zeros((D_IN, W), f32).at[:, :10].set(W1)
    b1e = jnp.zeros((1, W), f32).at[0, :10].set(b1)

    pad16 = lambda w: jnp.zeros((W, W), f32).at[:10, :10].set(w)
    wl1 = pad16(Wl1)
    wr1 = pad16(Wr1)
    wl2 = pad16(Wl2)
    wr2 = pad16(Wr2)
    bl1e = jnp.zeros((1, W), f32).at[0, :10].set(bl1)
    bl2e = jnp.zeros((1, W), f32).at[0, :10].set(bl2)
    w2e = jnp.zeros((W, 8), f32).at[:10, :3].set(W2)
    b2e = jnp.zeros((1, 8), f32).at[0, :3].set(b2)

    n0 = NS * C0 * CHUNK
    n1cap = NS * C1 * CHUNK
    pad_e = jnp.full((n0 + n1cap - N_EDGES,), DUMMY, jnp.int32)

    def _split(e):
        e0 = e[:n0].reshape(NS, C0, CHUNK)
        e0 = jnp.pad(e0, ((0, 0), (0, CMAX - C0), (0, 0)))
        e1 = jnp.concatenate([e[n0:], pad_e]).reshape(NS, C1, CHUNK)
        e1 = jnp.pad(e1, ((0, 0), (0, CMAX - C1), (0, 0)))
        return jnp.concatenate([e0, e1])

    srcp = _split(edge_index[0])
    dstp = _split(edge_index[1])

    hext = _tc1(x_pad, w1e, b1e)
    acc1, deg = _sc_agg(hext, srcp, dstp, True)
    h1, dgc = _tc2(acc1, deg, hext, wl1, bl1e, wr1)
    acc2 = _sc_agg(h1, srcp, dstp, False)
    out = _tc3(acc2, dgc, h1, wl2, bl2e, wr2, w2e, b2e)
    return out[:N_NODES, :3]


# speedup vs baseline: 1.1757x; 1.1757x over previous
import functools

import jax
import jax.numpy as jnp
from jax import lax
from jax.experimental import pallas as pl
from jax.experimental.pallas import tpu as pltpu
from jax.experimental.pallas import tpu_sc as plsc

N_NODES = 10000
N_EDGES = 320000
D_IN = 128
W = 16

NC = 2
NS = 16
NW = NC * NS

NP = 10240
ROWS_PER_TILE = NP // NS

CHUNK = 128
C0 = 64
C1 = 93
CMAX = max(C0, C1)
DUMMY = N_NODES

NBUF = 8
KAHEAD = 6

TCB1 = 2000
TCB = 2048
NTCB = NP // TCB


def _tc1_body(x_ref, w_ref, b_ref, o_ref):
    o_ref[...] = jnp.maximum(
        jnp.dot(x_ref[...], w_ref[...], preferred_element_type=jnp.float32)
        + b_ref[...], 0.0)


def _tc1(x, w1e, b1e):
    return pl.pallas_call(
        _tc1_body,
        grid=(N_NODES // TCB1,),
        in_specs=[
            pl.BlockSpec((TCB1, D_IN), lambda i: (i, 0)),
            pl.BlockSpec((D_IN, W), lambda i: (0, 0)),
            pl.BlockSpec((1, W), lambda i: (0, 0)),
        ],
        out_specs=pl.BlockSpec((TCB1, W), lambda i: (i, 0)),
        out_shape=jax.ShapeDtypeStruct((NP, W), jnp.float32),
    )(x, w1e, b1e)


def _tc2_body(acc_ref, deg_ref, hext_ref, wl_ref, bl_ref, wr_ref, h1_ref):
    a = acc_ref[0] + acc_ref[1]
    dgc = jnp.maximum(deg_ref[0] + deg_ref[1], 1.0).reshape(TCB, 1)
    mean = a / dgc
    h1_ref[...] = (
        jnp.dot(mean, wl_ref[...], preferred_element_type=jnp.float32)
        + bl_ref[...]
        + jnp.dot(hext_ref[...], wr_ref[...],
                  preferred_element_type=jnp.float32))


def _tc2(acc1, deg, hext, wl1, bl1, wr1):
    return pl.pallas_call(
        _tc2_body,
        grid=(NTCB,),
        in_specs=[
            pl.BlockSpec((NC, TCB, W), lambda i: (0, i, 0)),
            pl.BlockSpec((NC, TCB), lambda i: (0, i)),
            pl.BlockSpec((TCB, W), lambda i: (i, 0)),
            pl.BlockSpec((W, W), lambda i: (0, 0)),
            pl.BlockSpec((1, W), lambda i: (0, 0)),
            pl.BlockSpec((W, W), lambda i: (0, 0)),
        ],
        out_specs=pl.BlockSpec((TCB, W), lambda i: (i, 0)),
        out_shape=jax.ShapeDtypeStruct((NP, W), jnp.float32),
    )(acc1, deg, hext, wl1, bl1, wr1)


def _tc3_body(acc_ref, deg_ref, h1_ref, wl_ref, bl_ref, wr_ref, w2_ref,
              b2_ref, o_ref):
    a = acc_ref[0] + acc_ref[1]
    dgc = jnp.maximum(deg_ref[0] + deg_ref[1], 1.0).reshape(TCB, 1)
    mean = a / dgc
    h2 = (jnp.dot(mean, wl_ref[...], preferred_element_type=jnp.float32)
          + bl_ref[...]
          + jnp.dot(h1_ref[...], wr_ref[...],
                    preferred_element_type=jnp.float32))
    o_ref[...] = (jnp.dot(h2, w2_ref[...], preferred_element_type=jnp.float32)
                  + b2_ref[...])


def _tc3(acc2, deg, h1, wl2, bl2, wr2, w2e, b2e):
    return pl.pallas_call(
        _tc3_body,
        grid=(NTCB,),
        in_specs=[
            pl.BlockSpec((NC, TCB, W), lambda i: (0, i, 0)),
            pl.BlockSpec((NC, TCB), lambda i: (0, i)),
            pl.BlockSpec((TCB, W), lambda i: (i, 0)),
            pl.BlockSpec((W, W), lambda i: (0, 0)),
            pl.BlockSpec((1, W), lambda i: (0, 0)),
            pl.BlockSpec((W, W), lambda i: (0, 0)),
            pl.BlockSpec((W, 8), lambda i: (0, 0)),
            pl.BlockSpec((1, 8), lambda i: (0, 0)),
        ],
        out_specs=pl.BlockSpec((TCB, 8), lambda i: (i, 0)),
        out_shape=jax.ShapeDtypeStruct((NP, 8), jnp.float32),
    )(acc2, deg, h1, wl2, bl2, wr2, w2e, b2e)


def _sc_agg_body(with_deg, tab_hbm, src_hbm, dst_hbm, *rest):
    if with_deg:
        (out_hbm, dout_hbm, src_v, dst_v, rows_v, ones_v, zdeg_v, zrow_v,
         acc_sh, deg_sh, gsem, ssem, dsem, isem) = rest
    else:
        (out_hbm, src_v, dst_v, rows_v, zrow_v,
         acc_sh, gsem, ssem, isem) = rest
    cid = lax.axis_index("c")
    sid = lax.axis_index("s")
    wid = cid * NS + sid

    pltpu.async_copy(src_hbm.at[wid], src_v, isem).wait()
    pltpu.async_copy(dst_hbm.at[wid], dst_v, isem).wait()

    def _z(i, c):
        zrow_v[i] = jnp.zeros((W,), jnp.float32)
        return c
    lax.fori_loop(0, ROWS_PER_TILE, _z, 0)
    if with_deg:
        def _z1(i, c):
            zdeg_v[pl.ds(i * W, W)] = jnp.zeros((W,), jnp.float32)
            return c
        lax.fori_loop(0, ROWS_PER_TILE // W, _z1, 0)
        for i in range(CHUNK // W):
            ones_v[pl.ds(i * W, W)] = jnp.ones((W,), jnp.float32)
    row0 = sid * ROWS_PER_TILE
    pltpu.sync_copy(zrow_v, acc_sh.at[pl.ds(row0, ROWS_PER_TILE)])
    if with_deg:
        pltpu.sync_copy(zdeg_v, deg_sh.at[pl.ds(row0, ROWS_PER_TILE)])
    plsc.subcore_barrier()

    def _gather(j, b):
        pltpu.async_copy(tab_hbm.at[src_v.at[j]], rows_v.at[b], gsem.at[b])

    def _gather_wait(b):
        pltpu.make_async_copy(tab_hbm.at[src_v.at[0]], rows_v.at[b],
                              gsem.at[b]).wait()

    def _scatter(j, b):
        pltpu.async_copy(rows_v.at[b], acc_sh.at[dst_v.at[j]],
                         ssem.at[b], add=True)
        if with_deg:
            pltpu.async_copy(ones_v, deg_sh.at[dst_v.at[j]],
                             dsem.at[b], add=True)

    def _scatter_wait(b):
        pltpu.make_async_copy(rows_v.at[b], acc_sh.at[dst_v.at[0]],
                              ssem.at[b]).wait()
        if with_deg:
            pltpu.make_async_copy(ones_v, deg_sh.at[dst_v.at[0]],
                                  dsem.at[b]).wait()

    nche = lax.select(cid == 0, C0, C1)
    for j in range(KAHEAD):
        _gather(j, j)

    def _step(j, c):
        b = lax.rem(j, NBUF)
        jf = j + KAHEAD
        bf = lax.rem(jf, NBUF)

        @pl.when(jf < nche)
        def _prefetch():
            @pl.when(jf >= NBUF)
            def _drain():
                _scatter_wait(bf)
            _gather(jf, bf)

        _gather_wait(b)
        _scatter(j, b)
        return c

    lax.fori_loop(0, nche, _step, 0)

    for t in range(NBUF):
        _scatter_wait(t)
    plsc.subcore_barrier()
    pltpu.sync_copy(acc_sh.at[pl.ds(row0, ROWS_PER_TILE)],
                    out_hbm.at[cid, pl.ds(row0, ROWS_PER_TILE)])
    if with_deg:
        pltpu.sync_copy(deg_sh.at[pl.ds(row0, ROWS_PER_TILE)],
                        dout_hbm.at[cid, pl.ds(row0, ROWS_PER_TILE)])


def _sc_agg(table, srcp, dstp, with_deg):
    mesh = plsc.VectorSubcoreMesh(core_axis_name="c", subcore_axis_name="s")
    acc_t = jax.ShapeDtypeStruct((NC, NP, W), jnp.float32)
    if with_deg:
        out_type = (acc_t, jax.ShapeDtypeStruct((NC, NP), jnp.float32))
    else:
        out_type = acc_t
    scratch = [
        pltpu.VMEM((CMAX, CHUNK), jnp.int32),
        pltpu.VMEM((CMAX, CHUNK), jnp.int32),
        pltpu.VMEM((NBUF, CHUNK, W), jnp.float32),
    ]
    if with_deg:
        scratch.append(pltpu.VMEM((CHUNK,), jnp.float32))
        scratch.append(pltpu.VMEM((ROWS_PER_TILE,), jnp.float32))
    scratch.append(pltpu.VMEM((ROWS_PER_TILE, W), jnp.float32))
    scratch.append(pltpu.VMEM_SHARED((NP, W), jnp.float32))
    if with_deg:
        scratch.append(pltpu.VMEM_SHARED((NP,), jnp.float32))
    scratch.append(pltpu.SemaphoreType.DMA((NBUF,)))
    scratch.append(pltpu.SemaphoreType.DMA((NBUF,)))
    if with_deg:
        scratch.append(pltpu.SemaphoreType.DMA((NBUF,)))
    scratch.append(pltpu.SemaphoreType.DMA)
    kfn = pl.kernel(
        functools.partial(_sc_agg_body, with_deg),
        out_type=out_type,
        mesh=mesh,
        scratch_types=scratch,
        compiler_params=pltpu.CompilerParams(use_tc_tiling_on_sc=False),
    )
    return kfn(table, srcp, dstp)


def kernel(x, edge_index, W1, b1, Wl1, bl1, Wr1, Wl2, bl2, Wr2, W2, b2):
    f32 = jnp.float32

    w1e = jnp.zeros((D_IN, W), f32).at[:, :10].set(W1)
    b1e = jnp.zeros((1, W), f32).at[0, :10].set(b1)

    pad16 = lambda w: jnp.zeros((W, W), f32).at[:10, :10].set(w)
    wl1 = pad16(Wl1)
    wr1 = pad16(Wr1)
    wl2 = pad16(Wl2)
    wr2 = pad16(Wr2)
    bl1e = jnp.zeros((1, W), f32).at[0, :10].set(bl1)
    bl2e = jnp.zeros((1, W), f32).at[0, :10].set(bl2)
    w2e = jnp.zeros((W, 8), f32).at[:10, :3].set(W2)
    b2e = jnp.zeros((1, 8), f32).at[0, :3].set(b2)

    n0 = NS * C0 * CHUNK
    n1cap = NS * C1 * CHUNK
    pad_e = jnp.full((n0 + n1cap - N_EDGES,), DUMMY, jnp.int32)

    def _split(e):
        e0 = e[:n0].reshape(NS, C0, CHUNK)
        e0 = jnp.pad(e0, ((0, 0), (0, CMAX - C0), (0, 0)))
        e1 = jnp.concatenate([e[n0:], pad_e]).reshape(NS, C1, CHUNK)
        e1 = jnp.pad(e1, ((0, 0), (0, CMAX - C1), (0, 0)))
        return jnp.concatenate([e0, e1])

    srcp = _split(edge_index[0])
    dstp = _split(edge_index[1])

    hext = _tc1(x, w1e, b1e)
    acc1, deg = _sc_agg(hext, srcp, dstp, True)
    h1 = _tc2(acc1, deg, hext, wl1, bl1e, wr1)
    acc2 = _sc_agg(h1, srcp, dstp, False)
    out = _tc3(acc2, deg, h1, wl2, bl2e, wr2, w2e, b2e)
    return out[:N_NODES, :3]


# speedup vs baseline: 1.2734x; 1.0832x over previous
import functools

import jax
import jax.numpy as jnp
from jax import lax
from jax.experimental import pallas as pl
from jax.experimental.pallas import tpu as pltpu
from jax.experimental.pallas import tpu_sc as plsc

N_NODES = 10000
N_EDGES = 320000
D_IN = 128
W = 16

NC = 2
NS = 16
NW = NC * NS

NP = 10240
ROWS_PER_TILE = NP // NS

CHUNK = 128
C0 = 64
C1 = 93
CMAX = max(C0, C1)
DUMMY = N_NODES

NBUF = 8
KAHEAD = 6

TCB1 = 2000
TCB = 2048
NTCB = NP // TCB


def _tc1_body(x_ref, w_ref, b_ref, o_ref):
    o_ref[...] = jnp.maximum(
        jnp.dot(x_ref[...], w_ref[...], preferred_element_type=jnp.float32)
        + b_ref[...], 0.0)


def _tc1(x, w1e, b1e):
    return pl.pallas_call(
        _tc1_body,
        grid=(N_NODES // TCB1,),
        in_specs=[
            pl.BlockSpec((TCB1, D_IN), lambda i: (i, 0)),
            pl.BlockSpec((D_IN, W), lambda i: (0, 0)),
            pl.BlockSpec((1, W), lambda i: (0, 0)),
        ],
        out_specs=pl.BlockSpec((TCB1, W), lambda i: (i, 0)),
        out_shape=jax.ShapeDtypeStruct((NP, W), jnp.float32),
    )(x, w1e, b1e)


def _tc2_body(acc_ref, deg_ref, hext_ref, wl_ref, bl_ref, wr_ref, h1_ref):
    a = acc_ref[0] + acc_ref[1]
    dgc = jnp.maximum(deg_ref[0] + deg_ref[1], 1.0).reshape(TCB, 1)
    mean = a / dgc
    h1_ref[...] = (
        jnp.dot(mean, wl_ref[...], preferred_element_type=jnp.float32)
        + bl_ref[...]
        + jnp.dot(hext_ref[...], wr_ref[...],
                  preferred_element_type=jnp.float32))


def _tc2(acc1, deg, hext, wl1, bl1, wr1):
    return pl.pallas_call(
        _tc2_body,
        grid=(NTCB,),
        in_specs=[
            pl.BlockSpec((NC, TCB, W), lambda i: (0, i, 0)),
            pl.BlockSpec((NC, TCB), lambda i: (0, i)),
            pl.BlockSpec((TCB, W), lambda i: (i, 0)),
            pl.BlockSpec((W, W), lambda i: (0, 0)),
            pl.BlockSpec((1, W), lambda i: (0, 0)),
            pl.BlockSpec((W, W), lambda i: (0, 0)),
        ],
        out_specs=pl.BlockSpec((TCB, W), lambda i: (i, 0)),
        out_shape=jax.ShapeDtypeStruct((NP, W), jnp.float32),
    )(acc1, deg, hext, wl1, bl1, wr1)


def _tc3_body(acc_ref, deg_ref, h1_ref, wl_ref, bl_ref, wr_ref, w2_ref,
              b2_ref, o_ref):
    a = acc_ref[0] + acc_ref[1]
    dgc = jnp.maximum(deg_ref[0] + deg_ref[1], 1.0).reshape(TCB, 1)
    mean = a / dgc
    h2 = (jnp.dot(mean, wl_ref[...], preferred_element_type=jnp.float32)
          + bl_ref[...]
          + jnp.dot(h1_ref[...], wr_ref[...],
                    preferred_element_type=jnp.float32))
    o_ref[...] = (jnp.dot(h2, w2_ref[...], preferred_element_type=jnp.float32)
                  + b2_ref[...])


def _tc3(acc2, deg, h1, wl2, bl2, wr2, w2e, b2e):
    return pl.pallas_call(
        _tc3_body,
        grid=(NTCB,),
        in_specs=[
            pl.BlockSpec((NC, TCB, W), lambda i: (0, i, 0)),
            pl.BlockSpec((NC, TCB), lambda i: (0, i)),
            pl.BlockSpec((TCB, W), lambda i: (i, 0)),
            pl.BlockSpec((W, W), lambda i: (0, 0)),
            pl.BlockSpec((1, W), lambda i: (0, 0)),
            pl.BlockSpec((W, W), lambda i: (0, 0)),
            pl.BlockSpec((W, 8), lambda i: (0, 0)),
            pl.BlockSpec((1, 8), lambda i: (0, 0)),
        ],
        out_specs=pl.BlockSpec((TCB, 8), lambda i: (i, 0)),
        out_shape=jax.ShapeDtypeStruct((NP, 8), jnp.float32),
    )(acc2, deg, h1, wl2, bl2, wr2, w2e, b2e)


def _sc_agg_body(with_deg, tab_hbm, eidx_hbm, *rest):
    if with_deg:
        (out_hbm, dout_hbm, src_v, dst_v, rows_v, ones_v, zdeg_v, zrow_v,
         acc_sh, deg_sh, gsem, ssem, dsem, isem) = rest
    else:
        (out_hbm, src_v, dst_v, rows_v, zrow_v,
         acc_sh, gsem, ssem, isem) = rest
    cid = lax.axis_index("c")
    sid = lax.axis_index("s")

    base = lax.select(cid == 0, sid * C0, NS * C0 + sid * C1)
    pltpu.async_copy(eidx_hbm.at[0, pl.ds(base, CMAX)], src_v, isem).wait()
    pltpu.async_copy(eidx_hbm.at[1, pl.ds(base, CMAX)], dst_v, isem).wait()

    def _z(i, c):
        zrow_v[i] = jnp.zeros((W,), jnp.float32)
        return c
    lax.fori_loop(0, ROWS_PER_TILE, _z, 0)
    if with_deg:
        def _z1(i, c):
            zdeg_v[pl.ds(i * W, W)] = jnp.zeros((W,), jnp.float32)
            return c
        lax.fori_loop(0, ROWS_PER_TILE // W, _z1, 0)
        for i in range(CHUNK // W):
            ones_v[pl.ds(i * W, W)] = jnp.ones((W,), jnp.float32)
    row0 = sid * ROWS_PER_TILE
    pltpu.sync_copy(zrow_v, acc_sh.at[pl.ds(row0, ROWS_PER_TILE)])
    if with_deg:
        pltpu.sync_copy(zdeg_v, deg_sh.at[pl.ds(row0, ROWS_PER_TILE)])
    plsc.subcore_barrier()

    def _gather(j, b):
        pltpu.async_copy(tab_hbm.at[src_v.at[j]], rows_v.at[b], gsem.at[b])

    def _gather_wait(b):
        pltpu.make_async_copy(tab_hbm.at[src_v.at[0]], rows_v.at[b],
                              gsem.at[b]).wait()

    def _scatter(j, b):
        pltpu.async_copy(rows_v.at[b], acc_sh.at[dst_v.at[j]],
                         ssem.at[b], add=True)
        if with_deg:
            pltpu.async_copy(ones_v, deg_sh.at[dst_v.at[j]],
                             dsem.at[b], add=True)

    def _scatter_wait(b):
        pltpu.make_async_copy(rows_v.at[b], acc_sh.at[dst_v.at[0]],
                              ssem.at[b]).wait()
        if with_deg:
            pltpu.make_async_copy(ones_v, deg_sh.at[dst_v.at[0]],
                                  dsem.at[b]).wait()

    nche = lax.select(cid == 0, C0, C1)
    for j in range(KAHEAD):
        _gather(j, j)

    def _step(j, c):
        b = lax.rem(j, NBUF)
        jf = j + KAHEAD
        bf = lax.rem(jf, NBUF)

        @pl.when(jf < nche)
        def _prefetch():
            @pl.when(jf >= NBUF)
            def _drain():
                _scatter_wait(bf)
            _gather(jf, bf)

        _gather_wait(b)
        _scatter(j, b)
        return c

    lax.fori_loop(0, nche, _step, 0)

    for t in range(NBUF):
        _scatter_wait(t)
    plsc.subcore_barrier()
    pltpu.sync_copy(acc_sh.at[pl.ds(row0, ROWS_PER_TILE)],
                    out_hbm.at[cid, pl.ds(row0, ROWS_PER_TILE)])
    if with_deg:
        pltpu.sync_copy(deg_sh.at[pl.ds(row0, ROWS_PER_TILE)],
                        dout_hbm.at[cid, pl.ds(row0, ROWS_PER_TILE)])


def _sc_agg(table, eidx, with_deg):
    mesh = plsc.VectorSubcoreMesh(core_axis_name="c", subcore_axis_name="s")
    acc_t = jax.ShapeDtypeStruct((NC, NP, W), jnp.float32)
    if with_deg:
        out_type = (acc_t, jax.ShapeDtypeStruct((NC, NP), jnp.float32))
    else:
        out_type = acc_t
    scratch = [
        pltpu.VMEM((CMAX, CHUNK), jnp.int32),
        pltpu.VMEM((CMAX, CHUNK), jnp.int32),
        pltpu.VMEM((NBUF, CHUNK, W), jnp.float32),
    ]
    if with_deg:
        scratch.append(pltpu.VMEM((CHUNK,), jnp.float32))
        scratch.append(pltpu.VMEM((ROWS_PER_TILE,), jnp.float32))
    scratch.append(pltpu.VMEM((ROWS_PER_TILE, W), jnp.float32))
    scratch.append(pltpu.VMEM_SHARED((NP, W), jnp.float32))
    if with_deg:
        scratch.append(pltpu.VMEM_SHARED((NP,), jnp.float32))
    scratch.append(pltpu.SemaphoreType.DMA((NBUF,)))
    scratch.append(pltpu.SemaphoreType.DMA((NBUF,)))
    if with_deg:
        scratch.append(pltpu.SemaphoreType.DMA((NBUF,)))
    scratch.append(pltpu.SemaphoreType.DMA)
    kfn = pl.kernel(
        functools.partial(_sc_agg_body, with_deg),
        out_type=out_type,
        mesh=mesh,
        scratch_types=scratch,
        compiler_params=pltpu.CompilerParams(use_tc_tiling_on_sc=False),
    )
    return kfn(table, eidx)


def kernel(x, edge_index, W1, b1, Wl1, bl1, Wr1, Wl2, bl2, Wr2, W2, b2):
    f32 = jnp.float32

    w1e = jnp.zeros((D_IN, W), f32).at[:, :10].set(W1)
    b1e = jnp.zeros((1, W), f32).at[0, :10].set(b1)

    pad16 = lambda w: jnp.zeros((W, W), f32).at[:10, :10].set(w)
    wl1 = pad16(Wl1)
    wr1 = pad16(Wr1)
    wl2 = pad16(Wl2)
    wr2 = pad16(Wr2)
    bl1e = jnp.zeros((1, W), f32).at[0, :10].set(bl1)
    bl2e = jnp.zeros((1, W), f32).at[0, :10].set(bl2)
    w2e = jnp.zeros((W, 8), f32).at[:10, :3].set(W2)
    b2e = jnp.zeros((1, 8), f32).at[0, :3].set(b2)

    totch = NS * (C0 + C1)
    eidx = jnp.pad(edge_index, ((0, 0), (0, totch * CHUNK - N_EDGES)),
                   constant_values=DUMMY).reshape(2, totch, CHUNK)

    hext = _tc1(x, w1e, b1e)
    acc1, deg = _sc_agg(hext, eidx, True)
    h1 = _tc2(acc1, deg, hext, wl1, bl1e, wr1)
    acc2 = _sc_agg(h1, eidx, False)
    out = _tc3(acc2, deg, h1, wl2, bl2e, wr2, w2e, b2e)
    return out[:N_NODES, :3]


# speedup vs baseline: 1.3080x; 1.0271x over previous
import functools

import jax
import jax.numpy as jnp
from jax import lax
from jax.experimental import pallas as pl
from jax.experimental.pallas import tpu as pltpu
from jax.experimental.pallas import tpu_sc as plsc

N_NODES = 10000
N_EDGES = 320000
D_IN = 128
W = 16

NC = 2
NS = 16
NW = NC * NS

NP = 10240
ROWS_PER_TILE = NP // NS

CHUNK = 128
C0 = 76
C1 = 81
CMAX = max(C0, C1)
DUMMY = N_NODES

NBUF = 8
KAHEAD = 6

TCB1 = 2000
TCB = 2048
NTCB = NP // TCB


def _tc1_body(x_ref, w_ref, b_ref, o_ref):
    o_ref[...] = jnp.maximum(
        jnp.dot(x_ref[...], w_ref[...], preferred_element_type=jnp.float32)
        + b_ref[...], 0.0)


def _tc1(x, w1e, b1e):
    return pl.pallas_call(
        _tc1_body,
        grid=(N_NODES // TCB1,),
        in_specs=[
            pl.BlockSpec((TCB1, D_IN), lambda i: (i, 0)),
            pl.BlockSpec((D_IN, W), lambda i: (0, 0)),
            pl.BlockSpec((1, W), lambda i: (0, 0)),
        ],
        out_specs=pl.BlockSpec((TCB1, W), lambda i: (i, 0)),
        out_shape=jax.ShapeDtypeStruct((NP, W), jnp.float32),
    )(x, w1e, b1e)


def _tc2_body(acc_ref, deg_ref, hext_ref, wl_ref, bl_ref, wr_ref, h1_ref):
    a = acc_ref[0] + acc_ref[1]
    dgc = jnp.maximum(deg_ref[0] + deg_ref[1], 1.0).reshape(TCB, 1)
    mean = a / dgc
    h1_ref[...] = (
        jnp.dot(mean, wl_ref[...], preferred_element_type=jnp.float32)
        + bl_ref[...]
        + jnp.dot(hext_ref[...], wr_ref[...],
                  preferred_element_type=jnp.float32))


def _tc2(acc1, deg, hext, wl1, bl1, wr1):
    return pl.pallas_call(
        _tc2_body,
        grid=(NTCB,),
        in_specs=[
            pl.BlockSpec((NC, TCB, W), lambda i: (0, i, 0)),
            pl.BlockSpec((NC, TCB), lambda i: (0, i)),
            pl.BlockSpec((TCB, W), lambda i: (i, 0)),
            pl.BlockSpec((W, W), lambda i: (0, 0)),
            pl.BlockSpec((1, W), lambda i: (0, 0)),
            pl.BlockSpec((W, W), lambda i: (0, 0)),
        ],
        out_specs=pl.BlockSpec((TCB, W), lambda i: (i, 0)),
        out_shape=jax.ShapeDtypeStruct((NP, W), jnp.float32),
    )(acc1, deg, hext, wl1, bl1, wr1)


def _tc3_body(acc_ref, deg_ref, h1_ref, wl_ref, bl_ref, wr_ref, w2_ref,
              b2_ref, o_ref):
    a = acc_ref[0] + acc_ref[1]
    dgc = jnp.maximum(deg_ref[0] + deg_ref[1], 1.0).reshape(TCB, 1)
    mean = a / dgc
    h2 = (jnp.dot(mean, wl_ref[...], preferred_element_type=jnp.float32)
          + bl_ref[...]
          + jnp.dot(h1_ref[...], wr_ref[...],
                    preferred_element_type=jnp.float32))
    o_ref[...] = (jnp.dot(h2, w2_ref[...], preferred_element_type=jnp.float32)
                  + b2_ref[...])


def _tc3(acc2, deg, h1, wl2, bl2, wr2, w2e, b2e):
    return pl.pallas_call(
        _tc3_body,
        grid=(NTCB,),
        in_specs=[
            pl.BlockSpec((NC, TCB, W), lambda i: (0, i, 0)),
            pl.BlockSpec((NC, TCB), lambda i: (0, i)),
            pl.BlockSpec((TCB, W), lambda i: (i, 0)),
            pl.BlockSpec((W, W), lambda i: (0, 0)),
            pl.BlockSpec((1, W), lambda i: (0, 0)),
            pl.BlockSpec((W, W), lambda i: (0, 0)),
            pl.BlockSpec((W, 8), lambda i: (0, 0)),
            pl.BlockSpec((1, 8), lambda i: (0, 0)),
        ],
        out_specs=pl.BlockSpec((TCB, 8), lambda i: (i, 0)),
        out_shape=jax.ShapeDtypeStruct((NP, 8), jnp.float32),
    )(acc2, deg, h1, wl2, bl2, wr2, w2e, b2e)


def _sc_agg_body(with_deg, tab_hbm, eidx_hbm, *rest):
    if with_deg:
        (out_hbm, dout_hbm, src_v, dst_v, rows_v, ones_v, zdeg_v, zrow_v,
         acc_sh, deg_sh, gsem, ssem, dsem, isem) = rest
    else:
        (out_hbm, src_v, dst_v, rows_v, zrow_v,
         acc_sh, gsem, ssem, isem) = rest
    cid = lax.axis_index("c")
    sid = lax.axis_index("s")

    base = lax.select(cid == 0, sid * C0, NS * C0 + sid * C1)
    pltpu.async_copy(eidx_hbm.at[0, pl.ds(base, CMAX)], src_v, isem).wait()
    pltpu.async_copy(eidx_hbm.at[1, pl.ds(base, CMAX)], dst_v, isem).wait()

    def _z(i, c):
        zrow_v[i] = jnp.zeros((W,), jnp.float32)
        return c
    lax.fori_loop(0, ROWS_PER_TILE, _z, 0)
    if with_deg:
        def _z1(i, c):
            zdeg_v[pl.ds(i * W, W)] = jnp.zeros((W,), jnp.float32)
            return c
        lax.fori_loop(0, ROWS_PER_TILE // W, _z1, 0)
        for i in range(CHUNK // W):
            ones_v[pl.ds(i * W, W)] = jnp.ones((W,), jnp.float32)
    row0 = sid * ROWS_PER_TILE
    pltpu.sync_copy(zrow_v, acc_sh.at[pl.ds(row0, ROWS_PER_TILE)])
    if with_deg:
        pltpu.sync_copy(zdeg_v, deg_sh.at[pl.ds(row0, ROWS_PER_TILE)])
    plsc.subcore_barrier()

    def _gather(j, b):
        pltpu.async_copy(tab_hbm.at[src_v.at[j]], rows_v.at[b], gsem.at[b])

    def _gather_wait(b):
        pltpu.make_async_copy(tab_hbm.at[src_v.at[0]], rows_v.at[b],
                              gsem.at[b]).wait()

    def _scatter(j, b):
        pltpu.async_copy(rows_v.at[b], acc_sh.at[dst_v.at[j]],
                         ssem.at[b], add=True)
        if with_deg:
            pltpu.async_copy(ones_v, deg_sh.at[dst_v.at[j]],
                             dsem.at[b], add=True)

    def _scatter_wait(b):
        pltpu.make_async_copy(rows_v.at[b], acc_sh.at[dst_v.at[0]],
                              ssem.at[b]).wait()
        if with_deg:
            pltpu.make_async_copy(ones_v, deg_sh.at[dst_v.at[0]],
                                  dsem.at[b]).wait()

    nche = lax.select(cid == 0, C0, C1)
    for j in range(KAHEAD):
        _gather(j, j)

    def _step(j, c):
        b = lax.rem(j, NBUF)
        jf = j + KAHEAD
        bf = lax.rem(jf, NBUF)

        @pl.when(jf < nche)
        def _prefetch():
            @pl.when(jf >= NBUF)
            def _drain():
                _scatter_wait(bf)
            _gather(jf, bf)

        _gather_wait(b)
        _scatter(j, b)
        return c

    lax.fori_loop(0, nche, _step, 0)

    for t in range(NBUF):
        _scatter_wait(t)
    plsc.subcore_barrier()
    pltpu.sync_copy(acc_sh.at[pl.ds(row0, ROWS_PER_TILE)],
                    out_hbm.at[cid, pl.ds(row0, ROWS_PER_TILE)])
    if with_deg:
        pltpu.sync_copy(deg_sh.at[pl.ds(row0, ROWS_PER_TILE)],
                        dout_hbm.at[cid, pl.ds(row0, ROWS_PER_TILE)])


def _sc_agg(table, eidx, with_deg):
    mesh = plsc.VectorSubcoreMesh(core_axis_name="c", subcore_axis_name="s")
    acc_t = jax.ShapeDtypeStruct((NC, NP, W), jnp.float32)
    if with_deg:
        out_type = (acc_t, jax.ShapeDtypeStruct((NC, NP), jnp.float32))
    else:
        out_type = acc_t
    scratch = [
        pltpu.VMEM((CMAX, CHUNK), jnp.int32),
        pltpu.VMEM((CMAX, CHUNK), jnp.int32),
        pltpu.VMEM((NBUF, CHUNK, W), jnp.float32),
    ]
    if with_deg:
        scratch.append(pltpu.VMEM((CHUNK,), jnp.float32))
        scratch.append(pltpu.VMEM((ROWS_PER_TILE,), jnp.float32))
    scratch.append(pltpu.VMEM((ROWS_PER_TILE, W), jnp.float32))
    scratch.append(pltpu.VMEM_SHARED((NP, W), jnp.float32))
    if with_deg:
        scratch.append(pltpu.VMEM_SHARED((NP,), jnp.float32))
    scratch.append(pltpu.SemaphoreType.DMA((NBUF,)))
    scratch.append(pltpu.SemaphoreType.DMA((NBUF,)))
    if with_deg:
        scratch.append(pltpu.SemaphoreType.DMA((NBUF,)))
    scratch.append(pltpu.SemaphoreType.DMA)
    kfn = pl.kernel(
        functools.partial(_sc_agg_body, with_deg),
        out_type=out_type,
        mesh=mesh,
        scratch_types=scratch,
        compiler_params=pltpu.CompilerParams(use_tc_tiling_on_sc=False),
    )
    return kfn(table, eidx)


def kernel(x, edge_index, W1, b1, Wl1, bl1, Wr1, Wl2, bl2, Wr2, W2, b2):
    f32 = jnp.float32

    w1e = jnp.zeros((D_IN, W), f32).at[:, :10].set(W1)
    b1e = jnp.zeros((1, W), f32).at[0, :10].set(b1)

    pad16 = lambda w: jnp.zeros((W, W), f32).at[:10, :10].set(w)
    wl1 = pad16(Wl1)
    wr1 = pad16(Wr1)
    wl2 = pad16(Wl2)
    wr2 = pad16(Wr2)
    bl1e = jnp.zeros((1, W), f32).at[0, :10].set(bl1)
    bl2e = jnp.zeros((1, W), f32).at[0, :10].set(bl2)
    w2e = jnp.zeros((W, 8), f32).at[:10, :3].set(W2)
    b2e = jnp.zeros((1, 8), f32).at[0, :3].set(b2)

    totch = NS * (C0 + C1)
    eidx = jnp.pad(edge_index, ((0, 0), (0, totch * CHUNK - N_EDGES)),
                   constant_values=DUMMY).reshape(2, totch, CHUNK)

    hext = _tc1(x, w1e, b1e)
    acc1, deg = _sc_agg(hext, eidx, True)
    h1 = _tc2(acc1, deg, hext, wl1, bl1e, wr1)
    acc2 = _sc_agg(h1, eidx, False)
    out = _tc3(acc2, deg, h1, wl2, bl2e, wr2, w2e, b2e)
    return out[:N_NODES, :3]


# speedup vs baseline: 1.3092x; 1.0009x over previous
import functools

import jax
import jax.numpy as jnp
from jax import lax
from jax.experimental import pallas as pl
from jax.experimental.pallas import tpu as pltpu
from jax.experimental.pallas import tpu_sc as plsc

N_NODES = 10000
N_EDGES = 320000
D_IN = 128
W = 16

NC = 2
NS = 16
NW = NC * NS

NP = 10240
ROWS_PER_TILE = NP // NS

CHUNK = 128
C0 = 78
C1 = 79
CMAX = max(C0, C1)
DUMMY = N_NODES

NBUF = 8
KAHEAD = 6

TCB1 = 2000
TCB = 2048
NTCB = NP // TCB


def _tc1_body(x_ref, w_ref, b_ref, o_ref):
    o_ref[...] = jnp.maximum(
        jnp.dot(x_ref[...], w_ref[...], preferred_element_type=jnp.float32)
        + b_ref[...], 0.0)


def _tc1(x, w1e, b1e):
    return pl.pallas_call(
        _tc1_body,
        grid=(N_NODES // TCB1,),
        in_specs=[
            pl.BlockSpec((TCB1, D_IN), lambda i: (i, 0)),
            pl.BlockSpec((D_IN, W), lambda i: (0, 0)),
            pl.BlockSpec((1, W), lambda i: (0, 0)),
        ],
        out_specs=pl.BlockSpec((TCB1, W), lambda i: (i, 0)),
        out_shape=jax.ShapeDtypeStruct((NP, W), jnp.float32),
    )(x, w1e, b1e)


def _tc2_body(acc_ref, deg_ref, hext_ref, wl_ref, bl_ref, wr_ref, h1_ref):
    a = acc_ref[0] + acc_ref[1]
    dgc = jnp.maximum(deg_ref[0] + deg_ref[1], 1.0).reshape(TCB, 1)
    mean = a / dgc
    h1_ref[...] = (
        jnp.dot(mean, wl_ref[...], preferred_element_type=jnp.float32)
        + bl_ref[...]
        + jnp.dot(hext_ref[...], wr_ref[...],
                  preferred_element_type=jnp.float32))


def _tc2(acc1, deg, hext, wl1, bl1, wr1):
    return pl.pallas_call(
        _tc2_body,
        grid=(NTCB,),
        in_specs=[
            pl.BlockSpec((NC, TCB, W), lambda i: (0, i, 0)),
            pl.BlockSpec((NC, TCB), lambda i: (0, i)),
            pl.BlockSpec((TCB, W), lambda i: (i, 0)),
            pl.BlockSpec((W, W), lambda i: (0, 0)),
            pl.BlockSpec((1, W), lambda i: (0, 0)),
            pl.BlockSpec((W, W), lambda i: (0, 0)),
        ],
        out_specs=pl.BlockSpec((TCB, W), lambda i: (i, 0)),
        out_shape=jax.ShapeDtypeStruct((NP, W), jnp.float32),
    )(acc1, deg, hext, wl1, bl1, wr1)


def _tc3_body(acc_ref, deg_ref, h1_ref, wl_ref, bl_ref, wr_ref, w2_ref,
              b2_ref, o_ref):
    a = acc_ref[0] + acc_ref[1]
    dgc = jnp.maximum(deg_ref[0] + deg_ref[1], 1.0).reshape(TCB, 1)
    mean = a / dgc
    h2 = (jnp.dot(mean, wl_ref[...], preferred_element_type=jnp.float32)
          + bl_ref[...]
          + jnp.dot(h1_ref[...], wr_ref[...],
                    preferred_element_type=jnp.float32))
    o_ref[...] = (jnp.dot(h2, w2_ref[...], preferred_element_type=jnp.float32)
                  + b2_ref[...])


def _tc3(acc2, deg, h1, wl2, bl2, wr2, w2e, b2e):
    return pl.pallas_call(
        _tc3_body,
        grid=(NTCB,),
        in_specs=[
            pl.BlockSpec((NC, TCB, W), lambda i: (0, i, 0)),
            pl.BlockSpec((NC, TCB), lambda i: (0, i)),
            pl.BlockSpec((TCB, W), lambda i: (i, 0)),
            pl.BlockSpec((W, W), lambda i: (0, 0)),
            pl.BlockSpec((1, W), lambda i: (0, 0)),
            pl.BlockSpec((W, W), lambda i: (0, 0)),
            pl.BlockSpec((W, 8), lambda i: (0, 0)),
            pl.BlockSpec((1, 8), lambda i: (0, 0)),
        ],
        out_specs=pl.BlockSpec((TCB, 8), lambda i: (i, 0)),
        out_shape=jax.ShapeDtypeStruct((NP, 8), jnp.float32),
    )(acc2, deg, h1, wl2, bl2, wr2, w2e, b2e)


def _sc_agg_body(with_deg, tab_hbm, eidx_hbm, *rest):
    if with_deg:
        (out_hbm, dout_hbm, src_v, dst_v, rows_v, ones_v, zdeg_v, zrow_v,
         acc_sh, deg_sh, gsem, ssem, dsem, isem) = rest
    else:
        (out_hbm, src_v, dst_v, rows_v, zrow_v,
         acc_sh, gsem, ssem, isem) = rest
    cid = lax.axis_index("c")
    sid = lax.axis_index("s")

    base = lax.select(cid == 0, sid * C0, NS * C0 + sid * C1)
    pltpu.async_copy(eidx_hbm.at[0, pl.ds(base, CMAX)], src_v, isem).wait()
    pltpu.async_copy(eidx_hbm.at[1, pl.ds(base, CMAX)], dst_v, isem).wait()

    def _z(i, c):
        zrow_v[i] = jnp.zeros((W,), jnp.float32)
        return c
    lax.fori_loop(0, ROWS_PER_TILE, _z, 0)
    if with_deg:
        def _z1(i, c):
            zdeg_v[pl.ds(i * W, W)] = jnp.zeros((W,), jnp.float32)
            return c
        lax.fori_loop(0, ROWS_PER_TILE // W, _z1, 0)
        for i in range(CHUNK // W):
            ones_v[pl.ds(i * W, W)] = jnp.ones((W,), jnp.float32)
    row0 = sid * ROWS_PER_TILE
    pltpu.sync_copy(zrow_v, acc_sh.at[pl.ds(row0, ROWS_PER_TILE)])
    if with_deg:
        pltpu.sync_copy(zdeg_v, deg_sh.at[pl.ds(row0, ROWS_PER_TILE)])
    plsc.subcore_barrier()

    def _gather(j, b):
        pltpu.async_copy(tab_hbm.at[src_v.at[j]], rows_v.at[b], gsem.at[b])

    def _gather_wait(b):
        pltpu.make_async_copy(tab_hbm.at[src_v.at[0]], rows_v.at[b],
                              gsem.at[b]).wait()

    def _scatter(j, b):
        pltpu.async_copy(rows_v.at[b], acc_sh.at[dst_v.at[j]],
                         ssem.at[b], add=True)
        if with_deg:
            pltpu.async_copy(ones_v, deg_sh.at[dst_v.at[j]],
                             dsem.at[b], add=True)

    def _scatter_wait(b):
        pltpu.make_async_copy(rows_v.at[b], acc_sh.at[dst_v.at[0]],
                              ssem.at[b]).wait()
        if with_deg:
            pltpu.make_async_copy(ones_v, deg_sh.at[dst_v.at[0]],
                                  dsem.at[b]).wait()

    nche = lax.select(cid == 0, C0, C1)
    for j in range(KAHEAD):
        _gather(j, j)

    def _step(j, c):
        b = lax.rem(j, NBUF)
        jf = j + KAHEAD
        bf = lax.rem(jf, NBUF)

        @pl.when(jf < nche)
        def _prefetch():
            @pl.when(jf >= NBUF)
            def _drain():
                _scatter_wait(bf)
            _gather(jf, bf)

        _gather_wait(b)
        _scatter(j, b)
        return c

    lax.fori_loop(0, nche, _step, 0)

    for t in range(NBUF):
        _scatter_wait(t)
    plsc.subcore_barrier()
    pltpu.sync_copy(acc_sh.at[pl.ds(row0, ROWS_PER_TILE)],
                    out_hbm.at[cid, pl.ds(row0, ROWS_PER_TILE)])
    if with_deg:
        pltpu.sync_copy(deg_sh.at[pl.ds(row0, ROWS_PER_TILE)],
                        dout_hbm.at[cid, pl.ds(row0, ROWS_PER_TILE)])


def _sc_agg(table, eidx, with_deg):
    mesh = plsc.VectorSubcoreMesh(core_axis_name="c", subcore_axis_name="s")
    acc_t = jax.ShapeDtypeStruct((NC, NP, W), jnp.float32)
    if with_deg:
        out_type = (acc_t, jax.ShapeDtypeStruct((NC, NP), jnp.float32))
    else:
        out_type = acc_t
    scratch = [
        pltpu.VMEM((CMAX, CHUNK), jnp.int32),
        pltpu.VMEM((CMAX, CHUNK), jnp.int32),
        pltpu.VMEM((NBUF, CHUNK, W), jnp.float32),
    ]
    if with_deg:
        scratch.append(pltpu.VMEM((CHUNK,), jnp.float32))
        scratch.append(pltpu.VMEM((ROWS_PER_TILE,), jnp.float32))
    scratch.append(pltpu.VMEM((ROWS_PER_TILE, W), jnp.float32))
    scratch.append(pltpu.VMEM_SHARED((NP, W), jnp.float32))
    if with_deg:
        scratch.append(pltpu.VMEM_SHARED((NP,), jnp.float32))
    scratch.append(pltpu.SemaphoreType.DMA((NBUF,)))
    scratch.append(pltpu.SemaphoreType.DMA((NBUF,)))
    if with_deg:
        scratch.append(pltpu.SemaphoreType.DMA((NBUF,)))
    scratch.append(pltpu.SemaphoreType.DMA)
    kfn = pl.kernel(
        functools.partial(_sc_agg_body, with_deg),
        out_type=out_type,
        mesh=mesh,
        scratch_types=scratch,
        compiler_params=pltpu.CompilerParams(use_tc_tiling_on_sc=False),
    )
    return kfn(table, eidx)


def kernel(x, edge_index, W1, b1, Wl1, bl1, Wr1, Wl2, bl2, Wr2, W2, b2):
    f32 = jnp.float32

    w1e = jnp.zeros((D_IN, W), f32).at[:, :10].set(W1)
    b1e = jnp.zeros((1, W), f32).at[0, :10].set(b1)

    pad16 = lambda w: jnp.zeros((W, W), f32).at[:10, :10].set(w)
    wl1 = pad16(Wl1)
    wr1 = pad16(Wr1)
    wl2 = pad16(Wl2)
    wr2 = pad16(Wr2)
    bl1e = jnp.zeros((1, W), f32).at[0, :10].set(bl1)
    bl2e = jnp.zeros((1, W), f32).at[0, :10].set(bl2)
    w2e = jnp.zeros((W, 8), f32).at[:10, :3].set(W2)
    b2e = jnp.zeros((1, 8), f32).at[0, :3].set(b2)

    totch = NS * (C0 + C1)
    eidx = jnp.pad(edge_index, ((0, 0), (0, totch * CHUNK - N_EDGES)),
                   constant_values=DUMMY).reshape(2, totch, CHUNK)

    hext = _tc1(x, w1e, b1e)
    acc1, deg = _sc_agg(hext, eidx, True)
    h1 = _tc2(acc1, deg, hext, wl1, bl1e, wr1)
    acc2 = _sc_agg(h1, eidx, False)
    out = _tc3(acc2, deg, h1, wl2, bl2e, wr2, w2e, b2e)
    return out[:N_NODES, :3]


# speedup vs baseline: 1.4239x; 1.0877x over previous
import functools

import jax
import jax.numpy as jnp
from jax import lax
from jax.experimental import pallas as pl
from jax.experimental.pallas import tpu as pltpu
from jax.experimental.pallas import tpu_sc as plsc

N_NODES = 10000
N_EDGES = 320000
D_IN = 128
W = 16

NC = 2
NS = 16
NW = NC * NS

NP = 10240
ROWS_PER_TILE = NP // NS

CHUNK = 128
C0 = 78
C1 = 79
CMAX = max(C0, C1)
DUMMY = N_NODES

NBUF = 8
KAHEAD = 6

TCB1 = 2000
TCB = 2048
NTCB = NP // TCB


def _tc1_body(x_ref, w_ref, b_ref, o_ref):
    o_ref[...] = jnp.maximum(
        jnp.dot(x_ref[...], w_ref[...], preferred_element_type=jnp.float32)
        + b_ref[...], 0.0)


def _tc1(x, w1e, b1e):
    return pl.pallas_call(
        _tc1_body,
        grid=(N_NODES // TCB1,),
        in_specs=[
            pl.BlockSpec((TCB1, D_IN), lambda i: (i, 0)),
            pl.BlockSpec((D_IN, W), lambda i: (0, 0)),
            pl.BlockSpec((1, W), lambda i: (0, 0)),
        ],
        out_specs=pl.BlockSpec((TCB1, W), lambda i: (i, 0)),
        out_shape=jax.ShapeDtypeStruct((NP, W), jnp.float32),
    )(x, w1e, b1e)


def _tc2_body(acc_ref, deg_ref, hext_ref, wl_ref, bl_ref, wr_ref, h1_ref):
    a = acc_ref[0] + acc_ref[1]
    dgc = jnp.maximum(deg_ref[0] + deg_ref[1], 1.0).reshape(TCB, 1)
    mean = a / dgc
    h1_ref[...] = (
        jnp.dot(mean, wl_ref[...], preferred_element_type=jnp.float32)
        + bl_ref[...]
        + jnp.dot(hext_ref[...], wr_ref[...],
                  preferred_element_type=jnp.float32))


def _tc2(acc1, deg, hext, wl1, bl1, wr1):
    return pl.pallas_call(
        _tc2_body,
        grid=(NTCB,),
        in_specs=[
            pl.BlockSpec((NC, TCB, W), lambda i: (0, i, 0)),
            pl.BlockSpec((NC, TCB), lambda i: (0, i)),
            pl.BlockSpec((TCB, W), lambda i: (i, 0)),
            pl.BlockSpec((W, W), lambda i: (0, 0)),
            pl.BlockSpec((1, W), lambda i: (0, 0)),
            pl.BlockSpec((W, W), lambda i: (0, 0)),
        ],
        out_specs=pl.BlockSpec((TCB, W), lambda i: (i, 0)),
        out_shape=jax.ShapeDtypeStruct((NP, W), jnp.float32),
    )(acc1, deg, hext, wl1, bl1, wr1)


def _tc3_body(acc_ref, deg_ref, h1_ref, wl_ref, bl_ref, wr_ref, w2_ref,
              b2_ref, o_ref):
    a = acc_ref[0] + acc_ref[1]
    dgc = jnp.maximum(deg_ref[0] + deg_ref[1], 1.0).reshape(TCB, 1)
    mean = a / dgc
    h2 = (jnp.dot(mean, wl_ref[...], preferred_element_type=jnp.float32)
          + bl_ref[...]
          + jnp.dot(h1_ref[...], wr_ref[...],
                    preferred_element_type=jnp.float32))
    o_ref[...] = (jnp.dot(h2, w2_ref[...], preferred_element_type=jnp.float32)
                  + b2_ref[...])


def _tc3(acc2, deg, h1, wl2, bl2, wr2, w2e, b2e):
    return pl.pallas_call(
        _tc3_body,
        grid=(NTCB,),
        in_specs=[
            pl.BlockSpec((NC, TCB, W), lambda i: (0, i, 0)),
            pl.BlockSpec((NC, TCB), lambda i: (0, i)),
            pl.BlockSpec((TCB, W), lambda i: (i, 0)),
            pl.BlockSpec((W, W), lambda i: (0, 0)),
            pl.BlockSpec((1, W), lambda i: (0, 0)),
            pl.BlockSpec((W, W), lambda i: (0, 0)),
            pl.BlockSpec((W, 8), lambda i: (0, 0)),
            pl.BlockSpec((1, 8), lambda i: (0, 0)),
        ],
        out_specs=pl.BlockSpec((TCB, 8), lambda i: (i, 0)),
        out_shape=jax.ShapeDtypeStruct((NP, 8), jnp.float32),
    )(acc2, deg, h1, wl2, bl2, wr2, w2e, b2e)


def _sc_agg_body(with_deg, tab_hbm, eidx_hbm, *rest):
    if with_deg:
        (out_hbm, dout_hbm, src_v, dst_v, rows_v, ones_v, zdeg_v, zrow_v,
         acc_sh, deg_sh, tab_sh, gsem, ssem, dsem, isem) = rest
    else:
        (out_hbm, src_v, dst_v, rows_v, zrow_v,
         acc_sh, tab_sh, gsem, ssem, isem) = rest
    cid = lax.axis_index("c")
    sid = lax.axis_index("s")

    base = lax.select(cid == 0, sid * C0, NS * C0 + sid * C1)
    pltpu.async_copy(eidx_hbm.at[0, pl.ds(base, CMAX)], src_v, isem).wait()
    pltpu.async_copy(eidx_hbm.at[1, pl.ds(base, CMAX)], dst_v, isem).wait()

    def _z(i, c):
        zrow_v[i] = jnp.zeros((W,), jnp.float32)
        return c
    lax.fori_loop(0, ROWS_PER_TILE, _z, 0)
    if with_deg:
        def _z1(i, c):
            zdeg_v[pl.ds(i * W, W)] = jnp.zeros((W,), jnp.float32)
            return c
        lax.fori_loop(0, ROWS_PER_TILE // W, _z1, 0)
        for i in range(CHUNK // W):
            ones_v[pl.ds(i * W, W)] = jnp.ones((W,), jnp.float32)
    row0 = sid * ROWS_PER_TILE
    pltpu.sync_copy(tab_hbm.at[pl.ds(row0, ROWS_PER_TILE)],
                    tab_sh.at[pl.ds(row0, ROWS_PER_TILE)])
    pltpu.sync_copy(zrow_v, acc_sh.at[pl.ds(row0, ROWS_PER_TILE)])
    if with_deg:
        pltpu.sync_copy(zdeg_v, deg_sh.at[pl.ds(row0, ROWS_PER_TILE)])
    plsc.subcore_barrier()

    def _gather(j, b):
        pltpu.async_copy(tab_sh.at[src_v.at[j]], rows_v.at[b], gsem.at[b])

    def _gather_wait(b):
        pltpu.make_async_copy(tab_sh.at[src_v.at[0]], rows_v.at[b],
                              gsem.at[b]).wait()

    def _scatter(j, b):
        pltpu.async_copy(rows_v.at[b], acc_sh.at[dst_v.at[j]],
                         ssem.at[b], add=True)
        if with_deg:
            pltpu.async_copy(ones_v, deg_sh.at[dst_v.at[j]],
                             dsem.at[b], add=True)

    def _scatter_wait(b):
        pltpu.make_async_copy(rows_v.at[b], acc_sh.at[dst_v.at[0]],
                              ssem.at[b]).wait()
        if with_deg:
            pltpu.make_async_copy(ones_v, deg_sh.at[dst_v.at[0]],
                                  dsem.at[b]).wait()

    nche = lax.select(cid == 0, C0, C1)
    for j in range(KAHEAD):
        _gather(j, j)

    def _step(j, c):
        b = lax.rem(j, NBUF)
        jf = j + KAHEAD
        bf = lax.rem(jf, NBUF)

        @pl.when(jf < nche)
        def _prefetch():
            @pl.when(jf >= NBUF)
            def _drain():
                _scatter_wait(bf)
            _gather(jf, bf)

        _gather_wait(b)
        _scatter(j, b)
        return c

    lax.fori_loop(0, nche, _step, 0)

    for t in range(NBUF):
        _scatter_wait(t)
    plsc.subcore_barrier()
    pltpu.sync_copy(acc_sh.at[pl.ds(row0, ROWS_PER_TILE)],
                    out_hbm.at[cid, pl.ds(row0, ROWS_PER_TILE)])
    if with_deg:
        pltpu.sync_copy(deg_sh.at[pl.ds(row0, ROWS_PER_TILE)],
                        dout_hbm.at[cid, pl.ds(row0, ROWS_PER_TILE)])


def _sc_agg(table, eidx, with_deg):
    mesh = plsc.VectorSubcoreMesh(core_axis_name="c", subcore_axis_name="s")
    acc_t = jax.ShapeDtypeStruct((NC, NP, W), jnp.float32)
    if with_deg:
        out_type = (acc_t, jax.ShapeDtypeStruct((NC, NP), jnp.float32))
    else:
        out_type = acc_t
    scratch = [
        pltpu.VMEM((CMAX, CHUNK), jnp.int32),
        pltpu.VMEM((CMAX, CHUNK), jnp.int32),
        pltpu.VMEM((NBUF, CHUNK, W), jnp.float32),
    ]
    if with_deg:
        scratch.append(pltpu.VMEM((CHUNK,), jnp.float32))
        scratch.append(pltpu.VMEM((ROWS_PER_TILE,), jnp.float32))
    scratch.append(pltpu.VMEM((ROWS_PER_TILE, W), jnp.float32))
    scratch.append(pltpu.VMEM_SHARED((NP, W), jnp.float32))
    if with_deg:
        scratch.append(pltpu.VMEM_SHARED((NP,), jnp.float32))
    scratch.append(pltpu.VMEM_SHARED((NP, W), jnp.float32))
    scratch.append(pltpu.SemaphoreType.DMA((NBUF,)))
    scratch.append(pltpu.SemaphoreType.DMA((NBUF,)))
    if with_deg:
        scratch.append(pltpu.SemaphoreType.DMA((NBUF,)))
    scratch.append(pltpu.SemaphoreType.DMA)
    kfn = pl.kernel(
        functools.partial(_sc_agg_body, with_deg),
        out_type=out_type,
        mesh=mesh,
        scratch_types=scratch,
        compiler_params=pltpu.CompilerParams(use_tc_tiling_on_sc=False),
    )
    return kfn(table, eidx)


def kernel(x, edge_index, W1, b1, Wl1, bl1, Wr1, Wl2, bl2, Wr2, W2, b2):
    f32 = jnp.float32

    w1e = jnp.zeros((D_IN, W), f32).at[:, :10].set(W1)
    b1e = jnp.zeros((1, W), f32).at[0, :10].set(b1)

    pad16 = lambda w: jnp.zeros((W, W), f32).at[:10, :10].set(w)
    wl1 = pad16(Wl1)
    wr1 = pad16(Wr1)
    wl2 = pad16(Wl2)
    wr2 = pad16(Wr2)
    bl1e = jnp.zeros((1, W), f32).at[0, :10].set(bl1)
    bl2e = jnp.zeros((1, W), f32).at[0, :10].set(bl2)
    w2e = jnp.zeros((W, 8), f32).at[:10, :3].set(W2)
    b2e = jnp.zeros((1, 8), f32).at[0, :3].set(b2)

    totch = NS * (C0 + C1)
    eidx = jnp.pad(edge_index, ((0, 0), (0, totch * CHUNK - N_EDGES)),
                   constant_values=DUMMY).reshape(2, totch, CHUNK)

    hext = _tc1(x, w1e, b1e)
    acc1, deg = _sc_agg(hext, eidx, True)
    h1 = _tc2(acc1, deg, hext, wl1, bl1e, wr1)
    acc2 = _sc_agg(h1, eidx, False)
    out = _tc3(acc2, deg, h1, wl2, bl2e, wr2, w2e, b2e)
    return out[:N_NODES, :3]
